# Initial kernel scaffold; baseline (speedup 1.0000x reference)
import jax
import jax.numpy as jnp
from jax.experimental import pallas as pl


def kernel(G, idx):
    raise NotImplementedError("write your pallas kernel here")


# speedup vs baseline: 1.9443x; 1.9443x over previous
import functools

import jax
import jax.numpy as jnp
from jax import lax
from jax.experimental import pallas as pl
from jax.experimental.pallas import tpu as pltpu
from jax.experimental.pallas import tpu_sc as plsc

NC = 2
NS = 16
NW = NC * NS
LANES = 16

IDX_MINOR = 128
CHUNK_J = 16


def _make_gather(R, N, C, B):
    assert C == LANES and B % (2 * IDX_MINOR) == 0
    rows_per_w = B // 2
    j_per_w = rows_per_w // IDX_MINOR
    n_chunks = j_per_w // CHUNK_J

    mesh = plsc.VectorSubcoreMesh(
        core_axis_name="c", subcore_axis_name="s",
        num_cores=NC, num_subcores=NS)

    @functools.partial(
        pl.kernel,
        out_type=jax.ShapeDtypeStruct((R * B // IDX_MINOR, IDX_MINOR, C),
                                      jnp.float32),
        mesh=mesh,
        scratch_types=[
            pltpu.VMEM((j_per_w, IDX_MINOR), jnp.int32),
            pltpu.VMEM((CHUNK_J, IDX_MINOR, C), jnp.float32),
            pltpu.SemaphoreType.DMA,
        ],
        compiler_params=pltpu.CompilerParams(use_tc_tiling_on_sc=False),
    )
    def gather_kernel(table_hbm, idx_hbm, out_hbm, idx_v, rows_v, gsem):
        wid = lax.axis_index("s") * NC + lax.axis_index("c")
        r = wid // 2
        half = wid % 2
        pltpu.sync_copy(idx_hbm.at[pl.ds(half * j_per_w, j_per_w)], idx_v)
        roff = (r * N).astype(jnp.int32)

        def add_off(j, _):
            for i in range(IDX_MINOR // LANES):
                sl = pl.ds(i * LANES, LANES)
                idx_v[j, sl] = idx_v[j, sl] + roff
            return 0

        lax.fori_loop(0, j_per_w, add_off, 0)

        blk_base = (r * B + half * rows_per_w) // IDX_MINOR
        for chunk in range(n_chunks):
            copies = []
            for j in range(CHUNK_J):
                jj = chunk * CHUNK_J + j
                copies.append(pltpu.async_copy(
                    table_hbm.at[idx_v.at[jj]], rows_v.at[j], gsem))
            for cp in copies:
                cp.wait()
            pltpu.sync_copy(
                rows_v,
                out_hbm.at[pl.ds(blk_base + chunk * CHUNK_J, CHUNK_J)])

    return gather_kernel


def kernel(G, idx):
    R, N, C = G.shape
    B = idx.shape[0]
    table = G.reshape(R * N, C)
    idx2 = idx.reshape(B // IDX_MINOR, IDX_MINOR)
    out2 = _make_gather(R, N, C, B)(table, idx2)
    return out2.reshape(R, B, C)


# speedup vs baseline: 10.4576x; 5.3785x over previous
import functools

import jax
import jax.numpy as jnp
from jax import lax
from jax.experimental import pallas as pl
from jax.experimental.pallas import tpu as pltpu
from jax.experimental.pallas import tpu_sc as plsc

NC = 2
NS = 16
NW = NC * NS
LANES = 16

OUT_CHUNK = 4096


def _make_gather(P, N, B):
    rows_per_w = P // NW
    n_chunks = B // OUT_CHUNK

    mesh = plsc.VectorSubcoreMesh(
        core_axis_name="c", subcore_axis_name="s",
        num_cores=NC, num_subcores=NS)

    @functools.partial(
        pl.kernel,
        out_type=jax.ShapeDtypeStruct((P, B), jnp.float32),
        mesh=mesh,
        scratch_types=[
            pltpu.VMEM((B,), jnp.int32),
            pltpu.VMEM((N,), jnp.float32),
            pltpu.VMEM((2, OUT_CHUNK), jnp.float32),
            pltpu.SemaphoreType.DMA,
            pltpu.SemaphoreType.DMA,
        ],
        compiler_params=pltpu.CompilerParams(needs_layout_passes=False),
    )
    def gather_kernel(table_hbm, idx_hbm, out_hbm, idx_v, row_v, out_v,
                      wsem0, wsem1):
        wid = lax.axis_index("s") * NC + lax.axis_index("c")
        pltpu.sync_copy(idx_hbm, idx_v)
        wsems = (wsem0, wsem1)

        for k in range(rows_per_w):
            rc = wid * rows_per_w + k
            pltpu.sync_copy(table_hbm.at[rc], row_v)
            pending = [None, None]
            for chunk in range(n_chunks):
                buf = chunk % 2
                if pending[buf] is not None:
                    pending[buf].wait()
                    pending[buf] = None
                base = chunk * OUT_CHUNK

                def body(i, _, base=base, buf=buf):
                    o0 = i * (8 * LANES)
                    for u in range(8):
                        sl_in = pl.ds(base + o0 + u * LANES, LANES)
                        sl_out = pl.ds(o0 + u * LANES, LANES)
                        iv = idx_v[sl_in]
                        out_v[buf, sl_out] = plsc.load_gather(row_v, [iv])
                    return 0

                lax.fori_loop(0, OUT_CHUNK // (8 * LANES), body, 0)
                pending[buf] = pltpu.async_copy(
                    out_v.at[buf],
                    out_hbm.at[rc, pl.ds(base, OUT_CHUNK)],
                    wsems[buf])
            for p in pending:
                if p is not None:
                    p.wait()

    return gather_kernel


def kernel(G, idx):
    R, N, C = G.shape
    B = idx.shape[0]
    table = jnp.transpose(G, (0, 2, 1)).reshape(R * C, N)
    out2 = _make_gather(R * C, N, B)(table, idx)
    return jnp.transpose(out2.reshape(R, C, B), (0, 2, 1))


# speedup vs baseline: 18.3261x; 1.7524x over previous
import functools

import jax
import jax.numpy as jnp
from jax import lax
from jax.experimental import pallas as pl
from jax.experimental.pallas import tpu as pltpu
from jax.experimental.pallas import tpu_sc as plsc

NC = 2
NS = 16
NW = NC * NS
LANES = 16

OUT_CHUNK = 4096


def _make_gather(P, N, B):
    rows_per_w = P // NW
    n_chunks = B // OUT_CHUNK

    mesh = plsc.VectorSubcoreMesh(
        core_axis_name="c", subcore_axis_name="s",
        num_cores=NC, num_subcores=NS)

    @functools.partial(
        pl.kernel,
        out_type=jax.ShapeDtypeStruct((P, B), jnp.float32),
        mesh=mesh,
        scratch_types=[
            pltpu.VMEM((B,), jnp.int32),
            pltpu.VMEM((N,), jnp.float32),
            pltpu.VMEM((2, OUT_CHUNK), jnp.float32),
            pltpu.SemaphoreType.DMA,
            pltpu.SemaphoreType.DMA,
        ],
        compiler_params=pltpu.CompilerParams(needs_layout_passes=False),
    )
    def gather_kernel(table_hbm, idx_hbm, out_hbm, idx_v, row_v, out_v,
                      wsem0, wsem1):
        wid = lax.axis_index("s") * NC + lax.axis_index("c")
        pltpu.sync_copy(idx_hbm, idx_v)
        wsems = (wsem0, wsem1)

        for k in range(rows_per_w):
            rc = wid * rows_per_w + k
            pltpu.sync_copy(table_hbm.at[rc], row_v)
            pending = [None, None]
            for chunk in range(n_chunks):
                buf = chunk % 2
                if pending[buf] is not None:
                    pending[buf].wait()
                    pending[buf] = None
                base = chunk * OUT_CHUNK

                def body(g, base=base, buf=buf):
                    iv = idx_v[pl.ds(base + g * LANES, LANES)]
                    out_v[buf, pl.ds(g * LANES, LANES)] = (
                        plsc.load_gather(row_v, [iv]))

                plsc.parallel_loop(0, OUT_CHUNK // LANES, unroll=8)(body)
                pending[buf] = pltpu.async_copy(
                    out_v.at[buf],
                    out_hbm.at[rc, pl.ds(base, OUT_CHUNK)],
                    wsems[buf])
            for p in pending:
                if p is not None:
                    p.wait()

    return gather_kernel


def kernel(G, idx):
    R, N, C = G.shape
    B = idx.shape[0]
    table = jnp.transpose(G, (0, 2, 1)).reshape(R * C, N)
    out2 = _make_gather(R * C, N, B)(table, idx)
    return jnp.transpose(out2.reshape(R, C, B), (0, 2, 1))
